# Initial kernel scaffold; baseline (speedup 1.0000x reference)
import jax
import jax.numpy as jnp
from jax.experimental import pallas as pl


def kernel(label_map, label_embed_weight, pos_modulation):
    raise NotImplementedError("write your pallas kernel here")


# speedup vs baseline: 1.2509x; 1.2509x over previous
import functools

import jax
import jax.numpy as jnp
from jax import lax
from jax.experimental import pallas as pl
from jax.experimental.pallas import tpu as pltpu
from jax.experimental.pallas import tpu_sc as plsc

B, NUM_LABELS, LABEL_DIM, H, W = 16, 1024, 64, 128, 128
HW = H * W
NW = 32
P_PER_W = HW // NW
CHUNK = 256
N_CHUNKS = P_PER_W // CHUNK
LANES = 16


def _sc_body(idx_hbm, table_hbm, pos_hbm, out_hbm, table_v, idx_v, pos_v, out_v):
    wid = lax.axis_index("s") * 2 + lax.axis_index("c")
    pltpu.sync_copy(table_hbm, table_v)
    for half in range(N_CHUNKS):
        p0 = wid * P_PER_W + half * CHUNK
        pltpu.sync_copy(idx_hbm.at[:, pl.ds(p0, CHUNK)], idx_v)
        pltpu.sync_copy(pos_hbm.at[:, pl.ds(p0, CHUNK)], pos_v)
        for b in range(B):
            def c_loop(c, _):
                for v in range(CHUNK // LANES):
                    sl = pl.ds(v * LANES, LANES)
                    fi = idx_v[b, sl] * LABEL_DIM + c
                    row = plsc.load_gather(table_v, [fi])
                    out_v[c, sl] = row * pos_v[c, sl]
                return 0
            lax.fori_loop(0, LABEL_DIM, c_loop, 0)
            pltpu.sync_copy(out_v, out_hbm.at[b, :, pl.ds(p0, CHUNK)])


def kernel(label_map, label_embed_weight, pos_modulation):
    idx = label_map.reshape(B, HW)
    table_flat = label_embed_weight.reshape(NUM_LABELS * LABEL_DIM)
    pos = pos_modulation.reshape(LABEL_DIM, HW)

    run = functools.partial(
        pl.kernel,
        mesh=plsc.VectorSubcoreMesh(core_axis_name="c", subcore_axis_name="s"),
        out_type=jax.ShapeDtypeStruct((B, LABEL_DIM, HW), jnp.float32),
        scratch_types=[
            pltpu.VMEM((NUM_LABELS * LABEL_DIM,), jnp.float32),
            pltpu.VMEM((B, CHUNK), jnp.int32),
            pltpu.VMEM((LABEL_DIM, CHUNK), jnp.float32),
            pltpu.VMEM((LABEL_DIM, CHUNK), jnp.float32),
        ],
        compiler_params=pltpu.CompilerParams(needs_layout_passes=False),
    )(_sc_body)
    out = run(idx, table_flat, pos)
    return out.reshape(B, LABEL_DIM, H, W)


# speedup vs baseline: 3.0154x; 2.4105x over previous
import functools

import jax
import jax.numpy as jnp
from jax import lax
from jax.experimental import pallas as pl
from jax.experimental.pallas import tpu as pltpu
from jax.experimental.pallas import tpu_sc as plsc

B, NUM_LABELS, LABEL_DIM, H, W = 16, 1024, 64, 128, 128
HW = H * W
NW = 32
P_PER_W = HW // NW
CHUNK = 256
N_CHUNKS = P_PER_W // CHUNK
LANES = 16
NV = CHUNK // LANES


def _sc_body(idx_hbm, table_hbm, pos_hbm, out_hbm, table_v, idx_v, pos_v, out_v,
             sem):
    wid = lax.axis_index("s") * 2 + lax.axis_index("c")
    pltpu.sync_copy(table_hbm, table_v)
    for half in range(N_CHUNKS):
        p0 = wid * P_PER_W + half * CHUNK
        pltpu.sync_copy(idx_hbm.at[:, pl.ds(p0, CHUNK)], idx_v)
        pltpu.sync_copy(pos_hbm.at[:, pl.ds(p0, CHUNK)], pos_v)

        def b_body(b, _):
            parity = lax.rem(b, 2)

            @pl.when(b >= 2)
            def _wait_prev():
                pltpu.make_async_copy(
                    out_v.at[parity], out_hbm.at[b, :, pl.ds(p0, CHUNK)], sem
                ).wait()

            fib = [idx_v[b, pl.ds(v * LANES, LANES)] * LABEL_DIM
                   for v in range(NV)]

            @plsc.parallel_loop(0, LABEL_DIM, unroll=2)
            def _c_body(c):
                for v in range(NV):
                    sl = pl.ds(v * LANES, LANES)
                    row = plsc.load_gather(table_v, [fib[v] + c])
                    out_v[parity, c, sl] = row * pos_v[c, sl]

            pltpu.async_copy(
                out_v.at[parity], out_hbm.at[b, :, pl.ds(p0, CHUNK)], sem)
            return 0

        lax.fori_loop(0, B, b_body, 0)
        for tail in range(2):
            pltpu.make_async_copy(
                out_v.at[tail], out_hbm.at[tail, :, pl.ds(p0, CHUNK)], sem
            ).wait()


def kernel(label_map, label_embed_weight, pos_modulation):
    idx = label_map.reshape(B, HW)
    table_flat = label_embed_weight.reshape(NUM_LABELS * LABEL_DIM)
    pos = pos_modulation.reshape(LABEL_DIM, HW)

    run = functools.partial(
        pl.kernel,
        mesh=plsc.VectorSubcoreMesh(core_axis_name="c", subcore_axis_name="s"),
        out_type=jax.ShapeDtypeStruct((B, LABEL_DIM, HW), jnp.float32),
        scratch_types=[
            pltpu.VMEM((NUM_LABELS * LABEL_DIM,), jnp.float32),
            pltpu.VMEM((B, CHUNK), jnp.int32),
            pltpu.VMEM((LABEL_DIM, CHUNK), jnp.float32),
            pltpu.VMEM((2, LABEL_DIM, CHUNK), jnp.float32),
            pltpu.SemaphoreType.DMA,
        ],
        compiler_params=pltpu.CompilerParams(needs_layout_passes=False),
    )(_sc_body)
    out = run(idx, table_flat, pos)
    return out.reshape(B, LABEL_DIM, H, W)


# speedup vs baseline: 3.4689x; 1.1504x over previous
import functools

import jax
import jax.numpy as jnp
from jax import lax
from jax.experimental import pallas as pl
from jax.experimental.pallas import tpu as pltpu
from jax.experimental.pallas import tpu_sc as plsc

B, NUM_LABELS, LABEL_DIM, H, W = 16, 1024, 64, 128, 128
LANES = 16
BH = B // 2
RB = 8
CG = 16
NCG = LABEL_DIM // CG
VPR = W // LANES
NVB = 4


def _sc_body(idx_hbm, table_hbm, pos_hbm, out_hbm, table_v, idx_v, pos_v,
             out_v, sem):
    wid = lax.axis_index("s") * 2 + lax.axis_index("c")
    b0 = lax.rem(wid, 2) * BH
    h0 = lax.div(wid, 2) * RB
    pltpu.sync_copy(table_hbm, table_v)
    pltpu.sync_copy(idx_hbm.at[pl.ds(b0, BH), 0, pl.ds(h0, RB), :], idx_v)

    for cg in range(NCG):
        c0 = cg * CG
        pltpu.sync_copy(pos_hbm.at[0, pl.ds(c0, CG), pl.ds(h0, RB), :], pos_v)

        def b_body(b, _):
            parity = lax.rem(b, 2)

            @pl.when(b >= (2 if cg == 0 else 0))
            def _wait_prev():
                pltpu.make_async_copy(
                    out_v.at[parity],
                    out_hbm.at[b0 + b, pl.ds(c0, CG), pl.ds(h0, RB), :],
                    sem).wait()

            for vb in range(NVB):
                fib = [idx_v[b, 2 * vb + v // VPR,
                             pl.ds((v % VPR) * LANES, LANES)] * LABEL_DIM
                       for v in range(2 * VPR)]

                @plsc.parallel_loop(0, CG, unroll=2)
                def _c_body(c):
                    cv = jnp.full((LANES,), c0 + c, jnp.int32)
                    for v in range(2 * VPR):
                        r, off = 2 * vb + v // VPR, (v % VPR) * LANES
                        sl = pl.ds(off, LANES)
                        row = plsc.load_gather(table_v, [fib[v] + cv])
                        out_v[parity, c, r, sl] = row * pos_v[c, r, sl]

            pltpu.async_copy(
                out_v.at[parity],
                out_hbm.at[b0 + b, pl.ds(c0, CG), pl.ds(h0, RB), :], sem)
            return 0

        lax.fori_loop(0, BH, b_body, 0)

    for tail in range(2):
        pltpu.make_async_copy(
            out_v.at[tail], out_hbm.at[tail, pl.ds(0, CG), pl.ds(h0, RB), :],
            sem).wait()


def kernel(label_map, label_embed_weight, pos_modulation):
    table_flat = label_embed_weight.reshape(NUM_LABELS * LABEL_DIM)
    run = functools.partial(
        pl.kernel,
        mesh=plsc.VectorSubcoreMesh(core_axis_name="c", subcore_axis_name="s"),
        out_type=jax.ShapeDtypeStruct((B, LABEL_DIM, H, W), jnp.float32),
        scratch_types=[
            pltpu.VMEM((NUM_LABELS * LABEL_DIM,), jnp.float32),
            pltpu.VMEM((BH, RB, W), jnp.int32),
            pltpu.VMEM((CG, RB, W), jnp.float32),
            pltpu.VMEM((2, CG, RB, W), jnp.float32),
            pltpu.SemaphoreType.DMA,
        ],
        compiler_params=pltpu.CompilerParams(needs_layout_passes=False),
    )(_sc_body)
    return run(label_map, table_flat, pos_modulation)


# speedup vs baseline: 8.3467x; 2.4062x over previous
import functools

import jax
import jax.numpy as jnp
from jax import lax
from jax.experimental import pallas as pl
from jax.experimental.pallas import tpu as pltpu
from jax.experimental.pallas import tpu_sc as plsc

B, NUM_LABELS, LABEL_DIM, H, W = 16, 1024, 64, 128, 128
LANES = 16
BH = B // 2
RB = 8
CG = 16
NCG = LABEL_DIM // CG
VPR = W // LANES
NVB = 4


def _sc_body(idx_hbm, table_hbm, pos_hbm, out_hbm, table_v, idx_v, pos_v,
             out_v, sem):
    wid = lax.axis_index("s") * 2 + lax.axis_index("c")
    b0 = lax.rem(wid, 2) * BH
    h0 = lax.div(wid, 2) * RB
    pltpu.sync_copy(table_hbm, table_v)
    pltpu.sync_copy(idx_hbm.at[pl.ds(b0, BH), 0, pl.ds(h0, RB), :], idx_v)

    for cg in range(NCG):
        c0 = cg * CG
        pltpu.sync_copy(pos_hbm.at[0, pl.ds(c0, CG), pl.ds(h0, RB), :], pos_v)

        def b_body(b, _):
            parity = lax.rem(b, 2)

            @pl.when(b >= (2 if cg == 0 else 0))
            def _wait_prev():
                pltpu.make_async_copy(
                    out_v.at[parity],
                    out_hbm.at[b0 + b, pl.ds(c0, CG), pl.ds(h0, RB), :],
                    sem).wait()

            for vb in range(NVB):
                fib = [idx_v[b, 2 * vb + v // VPR,
                             pl.ds((v % VPR) * LANES, LANES)]
                       for v in range(2 * VPR)]

                @plsc.parallel_loop(0, CG, unroll=2)
                def _c_body(c):
                    cv = jnp.full((LANES,), (c0 + c) * NUM_LABELS, jnp.int32)
                    for v in range(2 * VPR):
                        r, off = 2 * vb + v // VPR, (v % VPR) * LANES
                        sl = pl.ds(off, LANES)
                        row = plsc.load_gather(table_v, [fib[v] + cv])
                        out_v[parity, c, r, sl] = row * pos_v[c, r, sl]

            pltpu.async_copy(
                out_v.at[parity],
                out_hbm.at[b0 + b, pl.ds(c0, CG), pl.ds(h0, RB), :], sem)
            return 0

        lax.fori_loop(0, BH, b_body, 0)

    for tail in range(2):
        pltpu.make_async_copy(
            out_v.at[tail], out_hbm.at[tail, pl.ds(0, CG), pl.ds(h0, RB), :],
            sem).wait()


def kernel(label_map, label_embed_weight, pos_modulation):
    table_flat = label_embed_weight.T.reshape(NUM_LABELS * LABEL_DIM)
    run = functools.partial(
        pl.kernel,
        mesh=plsc.VectorSubcoreMesh(core_axis_name="c", subcore_axis_name="s"),
        out_type=jax.ShapeDtypeStruct((B, LABEL_DIM, H, W), jnp.float32),
        scratch_types=[
            pltpu.VMEM((NUM_LABELS * LABEL_DIM,), jnp.float32),
            pltpu.VMEM((BH, RB, W), jnp.int32),
            pltpu.VMEM((CG, RB, W), jnp.float32),
            pltpu.VMEM((2, CG, RB, W), jnp.float32),
            pltpu.SemaphoreType.DMA,
        ],
        compiler_params=pltpu.CompilerParams(needs_layout_passes=False),
    )(_sc_body)
    return run(label_map, table_flat, pos_modulation)


# speedup vs baseline: 10.2699x; 1.2304x over previous
import functools

import jax
import jax.numpy as jnp
from jax import lax
from jax.experimental import pallas as pl
from jax.experimental.pallas import tpu as pltpu
from jax.experimental.pallas import tpu_sc as plsc

B, NUM_LABELS, LABEL_DIM, H, W = 16, 1024, 64, 128, 128
LANES = 16
BH = B // 2
RB = 8
CG = 8
NCG = LABEL_DIM // CG
VPR = W // LANES
NVB = 4


def _sc_body(idx_hbm, table_hbm, pos_hbm, out_hbm, table_v, idx_v, pos_v,
             out_v, sem, psem):
    wid = lax.axis_index("s") * 2 + lax.axis_index("c")
    b0 = lax.rem(wid, 2) * BH
    h0 = lax.div(wid, 2) * RB
    pltpu.sync_copy(table_hbm, table_v)
    pltpu.sync_copy(idx_hbm.at[pl.ds(b0, BH), 0, pl.ds(h0, RB), :], idx_v)
    pltpu.async_copy(
        pos_hbm.at[0, pl.ds(0, CG), pl.ds(h0, RB), :], pos_v.at[0], psem)

    def cg_body(cg, _):
        c0 = cg * CG
        pp = lax.rem(cg, 2)
        pltpu.make_async_copy(
            pos_hbm.at[0, pl.ds(c0, CG), pl.ds(h0, RB), :], pos_v.at[pp],
            psem).wait()

        @pl.when(cg + 1 < NCG)
        def _prefetch_pos():
            pltpu.async_copy(
                pos_hbm.at[0, pl.ds(c0 + CG, CG), pl.ds(h0, RB), :],
                pos_v.at[1 - pp], psem)

        def b_body(b, _):
            parity = lax.rem(b, 2)

            @pl.when(jnp.logical_or(cg > 0, b >= 2))
            def _wait_prev():
                pltpu.make_async_copy(
                    out_v.at[parity],
                    out_hbm.at[b0 + b, pl.ds(c0, CG), pl.ds(h0, RB), :],
                    sem).wait()

            for vb in range(NVB):
                fib = [idx_v[b, 2 * vb + v // VPR,
                             pl.ds((v % VPR) * LANES, LANES)]
                       for v in range(2 * VPR)]

                @plsc.parallel_loop(0, CG, unroll=4)
                def _c_body(c):
                    cv = jnp.full((LANES,), (c0 + c) * NUM_LABELS, jnp.int32)
                    for v in range(2 * VPR):
                        r, off = 2 * vb + v // VPR, (v % VPR) * LANES
                        sl = pl.ds(off, LANES)
                        row = plsc.load_gather(table_v, [fib[v] + cv])
                        out_v[parity, c, r, sl] = row * pos_v[pp, c, r, sl]

            pltpu.async_copy(
                out_v.at[parity],
                out_hbm.at[b0 + b, pl.ds(c0, CG), pl.ds(h0, RB), :], sem)
            return 0

        lax.fori_loop(0, BH, b_body, 0)
        return 0

    lax.fori_loop(0, NCG, cg_body, 0)

    for tail in range(2):
        pltpu.make_async_copy(
            out_v.at[tail], out_hbm.at[tail, pl.ds(0, CG), pl.ds(h0, RB), :],
            sem).wait()


def kernel(label_map, label_embed_weight, pos_modulation):
    table_flat = label_embed_weight.T.reshape(NUM_LABELS * LABEL_DIM)
    run = functools.partial(
        pl.kernel,
        mesh=plsc.VectorSubcoreMesh(core_axis_name="c", subcore_axis_name="s"),
        out_type=jax.ShapeDtypeStruct((B, LABEL_DIM, H, W), jnp.float32),
        scratch_types=[
            pltpu.VMEM((NUM_LABELS * LABEL_DIM,), jnp.float32),
            pltpu.VMEM((BH, RB, W), jnp.int32),
            pltpu.VMEM((2, CG, RB, W), jnp.float32),
            pltpu.VMEM((2, CG, RB, W), jnp.float32),
            pltpu.SemaphoreType.DMA,
            pltpu.SemaphoreType.DMA,
        ],
        compiler_params=pltpu.CompilerParams(needs_layout_passes=False),
    )(_sc_body)
    return run(label_map, table_flat, pos_modulation)


# speedup vs baseline: 10.2890x; 1.0019x over previous
import functools

import jax
import jax.numpy as jnp
from jax import lax
from jax.experimental import pallas as pl
from jax.experimental.pallas import tpu as pltpu
from jax.experimental.pallas import tpu_sc as plsc

B, NUM_LABELS, LABEL_DIM, H, W = 16, 1024, 64, 128, 128
LANES = 16
BH = B // 2
RB = 8
CG = 8
NCG = LABEL_DIM // CG
VPR = W // LANES
NVB = 4


def _sc_body(idx_hbm, table_hbm, pos_hbm, out_hbm, table_v, idx_v, pos_v,
             out_v, sem, psem):
    wid = lax.axis_index("s") * 2 + lax.axis_index("c")
    b0 = lax.rem(wid, 2) * BH
    h0 = lax.div(wid, 2) * RB
    pltpu.sync_copy(table_hbm, table_v)
    pltpu.sync_copy(idx_hbm.at[pl.ds(b0, BH), 0, pl.ds(h0, RB), :], idx_v)
    pltpu.async_copy(
        pos_hbm.at[0, pl.ds(0, CG), pl.ds(h0, RB), :], pos_v.at[0], psem)

    def cg_body(cg, _):
        c0 = cg * CG
        pp = lax.rem(cg, 2)
        pltpu.make_async_copy(
            pos_hbm.at[0, pl.ds(c0, CG), pl.ds(h0, RB), :], pos_v.at[pp],
            psem).wait()

        @pl.when(cg + 1 < NCG)
        def _prefetch_pos():
            pltpu.async_copy(
                pos_hbm.at[0, pl.ds(c0 + CG, CG), pl.ds(h0, RB), :],
                pos_v.at[1 - pp], psem)

        def b_body(b, _):
            parity = lax.rem(b, 2)

            @pl.when(jnp.logical_or(cg > 0, b >= 2))
            def _wait_prev():
                pltpu.make_async_copy(
                    out_v.at[parity],
                    out_hbm.at[b0 + b, pl.ds(c0, CG), pl.ds(h0, RB), :],
                    sem).wait()

            for vb in range(NVB):
                fib = [idx_v[b, 2 * vb + v // VPR,
                             pl.ds((v % VPR) * LANES, LANES)]
                       for v in range(2 * VPR)]

                @plsc.parallel_loop(0, CG, unroll=4)
                def _c_body(c):
                    cv = jnp.full((LANES,), (c0 + c) * NUM_LABELS, jnp.int32)
                    for v in range(2 * VPR):
                        r, off = 2 * vb + v // VPR, (v % VPR) * LANES
                        sl = pl.ds(off, LANES)
                        row = plsc.load_gather(table_v, [fib[v] + cv])
                        out_v[parity, c, r, sl] = row * pos_v[pp, c, r, sl]

            pltpu.async_copy(
                out_v.at[parity],
                out_hbm.at[b0 + b, pl.ds(c0, CG), pl.ds(h0, RB), :], sem)
            return 0

        lax.fori_loop(0, BH, b_body, 0)
        return 0

    lax.fori_loop(0, NCG, cg_body, 0)

    for tail in range(2):
        pltpu.make_async_copy(
            out_v.at[tail], out_hbm.at[tail, pl.ds(0, CG), pl.ds(h0, RB), :],
            sem).wait()


def kernel(label_map, label_embed_weight, pos_modulation):
    table_flat = label_embed_weight.T.reshape(NUM_LABELS * LABEL_DIM)
    run = functools.partial(
        pl.kernel,
        mesh=plsc.VectorSubcoreMesh(core_axis_name="c", subcore_axis_name="s"),
        out_type=jax.ShapeDtypeStruct((B, LABEL_DIM, H, W), jnp.float32),
        scratch_types=[
            pltpu.VMEM((NUM_LABELS * LABEL_DIM,), jnp.float32),
            pltpu.VMEM((BH, RB, W), jnp.int32),
            pltpu.VMEM((2, CG, RB, W), jnp.float32),
            pltpu.VMEM((2, CG, RB, W), jnp.float32),
            pltpu.SemaphoreType.DMA,
            pltpu.SemaphoreType.DMA,
        ],
        compiler_params=pltpu.CompilerParams(
            needs_layout_passes=False, skip_device_barrier=True),
    )(_sc_body)
    return run(label_map, table_flat, pos_modulation)


# speedup vs baseline: 10.3498x; 1.0059x over previous
import functools

import jax
import jax.numpy as jnp
from jax import lax
from jax.experimental import pallas as pl
from jax.experimental.pallas import tpu as pltpu
from jax.experimental.pallas import tpu_sc as plsc

B, NUM_LABELS, LABEL_DIM, H, W = 16, 1024, 64, 128, 128
LANES = 16
BH = B // 2
RB = 8
CG = 8
CPG = CG // 2
NCG = LABEL_DIM // CG
VPR = W // LANES
NVB = 4


def _sc_body(idx_hbm, table_hbm, pos_hbm, out_hbm, table_v, idx_v, pos_v,
             out_v, sem, psem):
    wid = lax.axis_index("s") * 2 + lax.axis_index("c")
    b0 = lax.rem(wid, 2) * BH
    h0 = lax.div(wid, 2) * RB
    pltpu.sync_copy(table_hbm, table_v)
    pltpu.sync_copy(idx_hbm.at[pl.ds(b0, BH), 0, pl.ds(h0, RB), :], idx_v)
    pltpu.async_copy(
        pos_hbm.at[0, pl.ds(0, CG), pl.ds(h0, RB), :], pos_v.at[0], psem)

    def cg_body(cg, _):
        c0 = cg * CG
        pp = lax.rem(cg, 2)
        pltpu.make_async_copy(
            pos_hbm.at[0, pl.ds(c0, CG), pl.ds(h0, RB), :], pos_v.at[pp],
            psem).wait()

        @pl.when(cg + 1 < NCG)
        def _prefetch_pos():
            pltpu.async_copy(
                pos_hbm.at[0, pl.ds(c0 + CG, CG), pl.ds(h0, RB), :],
                pos_v.at[1 - pp], psem)

        def b_body(b, _):
            parity = lax.rem(b, 2)

            @pl.when(jnp.logical_or(cg > 0, b >= 2))
            def _wait_prev():
                pltpu.make_async_copy(
                    out_v.at[parity],
                    out_hbm.at[b0 + b, pl.ds(c0, CG), pl.ds(h0, RB), :],
                    sem).wait()

            for vb in range(NVB):
                fib = [idx_v[b, 2 * vb + v // VPR,
                             pl.ds((v % VPR) * LANES, LANES)]
                       for v in range(2 * VPR)]

                @plsc.parallel_loop(0, CPG, unroll=CPG)
                def _cp_body(cp):
                    cpv = jnp.full(
                        (LANES,), (cg * CPG + cp) * NUM_LABELS, jnp.int32)
                    for v in range(2 * VPR):
                        r, off = 2 * vb + v // VPR, (v % VPR) * LANES
                        sl = pl.ds(off, LANES)
                        pair = plsc.load_gather(table_v, [fib[v] + cpv])
                        lo, hi = plsc.unpack(
                            plsc.bitcast(pair, jnp.bfloat16),
                            format=plsc.PackFormat.INTERLEAVED)
                        out_v[parity, 2 * cp, r, sl] = (
                            lo * pos_v[pp, 2 * cp, r, sl])
                        out_v[parity, 2 * cp + 1, r, sl] = (
                            hi * pos_v[pp, 2 * cp + 1, r, sl])

            pltpu.async_copy(
                out_v.at[parity],
                out_hbm.at[b0 + b, pl.ds(c0, CG), pl.ds(h0, RB), :], sem)
            return 0

        lax.fori_loop(0, BH, b_body, 0)
        return 0

    lax.fori_loop(0, NCG, cg_body, 0)

    for tail in range(2):
        pltpu.make_async_copy(
            out_v.at[tail], out_hbm.at[tail, pl.ds(0, CG), pl.ds(h0, RB), :],
            sem).wait()


def kernel(label_map, label_embed_weight, pos_modulation):
    wt = label_embed_weight.astype(jnp.bfloat16).T
    pairs = wt.reshape(LABEL_DIM // 2, 2, NUM_LABELS).transpose(0, 2, 1)
    packed = jax.lax.bitcast_convert_type(pairs, jnp.float32)
    table_packed = packed.reshape(LABEL_DIM // 2 * NUM_LABELS)

    run = functools.partial(
        pl.kernel,
        mesh=plsc.VectorSubcoreMesh(core_axis_name="c", subcore_axis_name="s"),
        out_type=jax.ShapeDtypeStruct((B, LABEL_DIM, H, W), jnp.float32),
        scratch_types=[
            pltpu.VMEM((LABEL_DIM // 2 * NUM_LABELS,), jnp.float32),
            pltpu.VMEM((BH, RB, W), jnp.int32),
            pltpu.VMEM((2, CG, RB, W), jnp.float32),
            pltpu.VMEM((2, CG, RB, W), jnp.float32),
            pltpu.SemaphoreType.DMA,
            pltpu.SemaphoreType.DMA,
        ],
        compiler_params=pltpu.CompilerParams(
            needs_layout_passes=False, skip_device_barrier=True),
    )(_sc_body)
    return run(label_map, table_packed, pos_modulation)


# speedup vs baseline: 11.5412x; 1.1151x over previous
import functools

import jax
import jax.numpy as jnp
from jax import lax
from jax.experimental import pallas as pl
from jax.experimental.pallas import tpu as pltpu
from jax.experimental.pallas import tpu_sc as plsc

B, NUM_LABELS, LABEL_DIM, H, W = 16, 1024, 64, 128, 128
LANES = 16
BH = B // 2
RB = 8
CG = 16
CPG = CG // 2
NCG = LABEL_DIM // CG
VPR = W // LANES
NVB = 4


def _sc_body(idx_hbm, table_hbm, pos_hbm, out_hbm, table_v, idx_v, pos_v,
             out_v, sem, psem):
    wid = lax.axis_index("s") * 2 + lax.axis_index("c")
    b0 = lax.rem(wid, 2) * BH
    h0 = lax.div(wid, 2) * RB
    pltpu.sync_copy(table_hbm, table_v)
    pltpu.sync_copy(idx_hbm.at[pl.ds(b0, BH), 0, pl.ds(h0, RB), :], idx_v)
    pltpu.async_copy(
        pos_hbm.at[0, pl.ds(0, CG), pl.ds(h0, RB), :], pos_v.at[0], psem)

    def cg_body(cg, _):
        c0 = cg * CG
        pp = lax.rem(cg, 2)
        pltpu.make_async_copy(
            pos_hbm.at[0, pl.ds(c0, CG), pl.ds(h0, RB), :], pos_v.at[pp],
            psem).wait()

        @pl.when(cg + 1 < NCG)
        def _prefetch_pos():
            pltpu.async_copy(
                pos_hbm.at[0, pl.ds(c0 + CG, CG), pl.ds(h0, RB), :],
                pos_v.at[1 - pp], psem)

        def b_body(b, _):
            parity = lax.rem(b, 2)

            @pl.when(jnp.logical_or(cg > 0, b >= 2))
            def _wait_prev():
                pltpu.make_async_copy(
                    out_v.at[parity],
                    out_hbm.at[b0 + b, pl.ds(c0, CG), pl.ds(h0, RB), :],
                    sem).wait()

            for vb in range(NVB):
                fib = [idx_v[b, 2 * vb + v // VPR,
                             pl.ds((v % VPR) * LANES, LANES)]
                       for v in range(2 * VPR)]

                @plsc.parallel_loop(0, CPG, unroll=4)
                def _cp_body(cp):
                    cpv = jnp.full(
                        (LANES,), (cg * CPG + cp) * NUM_LABELS, jnp.int32)
                    for v in range(2 * VPR):
                        r, off = 2 * vb + v // VPR, (v % VPR) * LANES
                        sl = pl.ds(off, LANES)
                        pair = plsc.load_gather(table_v, [fib[v] + cpv])
                        lo, hi = plsc.unpack(
                            plsc.bitcast(pair, jnp.bfloat16),
                            format=plsc.PackFormat.INTERLEAVED)
                        out_v[parity, 2 * cp, r, sl] = (
                            lo * pos_v[pp, 2 * cp, r, sl])
                        out_v[parity, 2 * cp + 1, r, sl] = (
                            hi * pos_v[pp, 2 * cp + 1, r, sl])

            pltpu.async_copy(
                out_v.at[parity],
                out_hbm.at[b0 + b, pl.ds(c0, CG), pl.ds(h0, RB), :], sem)
            return 0

        lax.fori_loop(0, BH, b_body, 0)
        return 0

    lax.fori_loop(0, NCG, cg_body, 0)

    for tail in range(2):
        pltpu.make_async_copy(
            out_v.at[tail], out_hbm.at[tail, pl.ds(0, CG), pl.ds(h0, RB), :],
            sem).wait()


def kernel(label_map, label_embed_weight, pos_modulation):
    wt = label_embed_weight.astype(jnp.bfloat16).T
    pairs = wt.reshape(LABEL_DIM // 2, 2, NUM_LABELS).transpose(0, 2, 1)
    packed = jax.lax.bitcast_convert_type(pairs, jnp.float32)
    table_packed = packed.reshape(LABEL_DIM // 2 * NUM_LABELS)

    run = functools.partial(
        pl.kernel,
        mesh=plsc.VectorSubcoreMesh(core_axis_name="c", subcore_axis_name="s"),
        out_type=jax.ShapeDtypeStruct((B, LABEL_DIM, H, W), jnp.float32),
        scratch_types=[
            pltpu.VMEM((LABEL_DIM // 2 * NUM_LABELS,), jnp.float32),
            pltpu.VMEM((BH, RB, W), jnp.int32),
            pltpu.VMEM((2, CG, RB, W), jnp.float32),
            pltpu.VMEM((2, CG, RB, W), jnp.float32),
            pltpu.SemaphoreType.DMA,
            pltpu.SemaphoreType.DMA,
        ],
        compiler_params=pltpu.CompilerParams(
            needs_layout_passes=False, skip_device_barrier=True),
    )(_sc_body)
    return run(label_map, table_packed, pos_modulation)
